# baseline (speedup 1.0000x reference)
import functools

import jax
import jax.numpy as jnp
from jax import lax
from jax.experimental import pallas as pl
from jax.experimental.pallas import tpu as pltpu
from jax.experimental.pallas import tpu_sc as plsc

_NBUF = 4
_PB = 4


def kernel(x, table):
    B, S = x.shape
    V, D = table.shape

    info = plsc.get_sparse_core_info()
    NC, NS = info.num_cores, info.num_subcores
    NW = NC * NS
    W = B // NW
    ST = S // 8
    NBT = B // 512

    assert W == 128 and D == 64 and S % 8 == 0 and B % 512 == 0
    assert (S - _NBUF) % _NBUF == 0 and S >= 2 * _NBUF

    x4 = x.astype(jnp.int32).reshape(NW, W, ST, 8).transpose(2, 0, 3, 1)

    mesh = plsc.VectorSubcoreMesh(core_axis_name="c", subcore_axis_name="s")

    @functools.partial(
        pl.kernel,
        mesh=mesh,
        compiler_params=pltpu.CompilerParams(
            use_tc_tiling_on_sc=False, needs_layout_passes=False
        ),
        out_type=jax.ShapeDtypeStruct((NBT, S // 2, 512, 2 * D), jnp.float32),
        scratch_types=[
            pltpu.VMEM((ST, 8, W), jnp.int32),
            pltpu.VMEM((_NBUF, W, D), jnp.float32),
            [pltpu.SemaphoreType.DMA] * _NBUF,
            [pltpu.SemaphoreType.DMA] * _NBUF,
        ],
    )
    def gather_sc(idx_hbm, tab_hbm, inter_hbm, idx_all, gbuf, gsems, osems):
        wid = lax.axis_index("s") * NC + lax.axis_index("c")
        bt = lax.shift_right_logical(wid, 2)
        bsub0 = lax.bitwise_and(wid, 3) * W

        pltpu.sync_copy(idx_hbm.at[:, wid], idx_all)

        def dst(g):
            return inter_hbm.at[
                bt, g >> 1, pl.ds(bsub0, W), pl.ds((g & 1) * D, D)
            ]

        def fire(g, b):
            pltpu.async_copy(
                tab_hbm.at[idx_all.at[g >> 3, g & 7]], gbuf.at[b], gsems[b]
            )

        def wait_g(b):
            pltpu.make_async_copy(tab_hbm.at[pl.ds(0, W)], gbuf.at[b], gsems[b]).wait()

        def fire_w(g, b):
            pltpu.async_copy(gbuf.at[b], dst(g), osems[b])

        def wait_w(b):
            pltpu.make_async_copy(gbuf.at[b], dst(0), osems[b]).wait()

        for b in range(_NBUF):
            fire(b, b)

        def body(i, carry):
            gg = i * _NBUF
            for b in range(_NBUF):
                g = gg + b
                wait_g(b)
                fire_w(g, b)
                wait_w(b)
                fire(g + _NBUF, b)
            return carry

        lax.fori_loop(0, (S - _NBUF) // _NBUF, body, 0)

        for b in range(_NBUF):
            g = S - _NBUF + b
            wait_g(b)
            fire_w(g, b)
            wait_w(b)

    def trans_tc(in_ref, out_ref):
        for q in range(_PB):
            blk = in_ref[0, q]
            out_ref[2 * q] = blk[:, :D].T
            out_ref[2 * q + 1] = blk[:, D:].T

    inter = gather_sc(x4, table)
    out_sdb = pl.pallas_call(
        trans_tc,
        grid=(NBT, S // (2 * _PB)),
        in_specs=[
            pl.BlockSpec((1, _PB, 512, 2 * D), lambda i, j: (i, j, 0, 0)),
        ],
        out_specs=pl.BlockSpec((2 * _PB, D, 512), lambda i, j: (j, 0, i)),
        out_shape=jax.ShapeDtypeStruct((S, D, B), jnp.float32),
    )(inter)
    return out_sdb.transpose(2, 0, 1)
